# Initial kernel scaffold; baseline (speedup 1.0000x reference)
import jax
import jax.numpy as jnp
from jax.experimental import pallas as pl


def kernel(h_V, h_E, edge_idx, W1_w, W1_b, W2_w, W2_b, W3_w, W3_b, d1_w, d1_b, d2_w, d2_b, ln1_g, ln1_b, ln2_g, ln2_b):
    raise NotImplementedError("write your pallas kernel here")


# speedup vs baseline: 2.1947x; 2.1947x over previous
import functools

import jax
import jax.numpy as jnp
from jax import lax
from jax.experimental import pallas as pl
from jax.experimental.pallas import tpu as pltpu
from jax.experimental.pallas import tpu_sc as plsc

N_NODES = 10000
N_EDGES = 320000
H = 128
NIN = 16
SCALE = 30.0

NC = 2
NS = 16
NW = NC * NS
EDGES_PER_WORKER = N_EDGES // NW
CHUNK = 80
NCHUNKS = EDGES_PER_WORKER // CHUNK


def _gelu(x):
    return 0.5 * x * (1.0 + lax.erf(x * (2.0 ** -0.5)))


BE = 1280


def _edge_mlp_body(xa_ref, xb_ref, w1a_ref, w1b_ref, b1_ref, w2_ref, b2_ref,
                   w3_ref, b3_ref, out_ref):
    xa = xa_ref[...]
    xb = xb_ref[...]
    m = _gelu(jnp.dot(xa, w1a_ref[...], preferred_element_type=jnp.float32)
              + jnp.dot(xb, w1b_ref[...], preferred_element_type=jnp.float32)
              + b1_ref[...])
    m = _gelu(jnp.dot(m, w2_ref[...], preferred_element_type=jnp.float32)
              + b2_ref[...])
    out_ref[...] = (jnp.dot(m, w3_ref[...], preferred_element_type=jnp.float32)
                    + b3_ref[...])


def _edge_mlp(h_E, W1_w, W1_b, W2_w, W2_b, W3_w, W3_b):
    xa = h_E[:, :H]
    xb = h_E[:, H:]
    w1a = W1_w[:H]
    w1b = W1_w[H:]
    grid = (N_EDGES // BE,)
    full = lambda s: pl.BlockSpec(s, lambda i: (0, 0))
    return pl.pallas_call(
        _edge_mlp_body,
        grid=grid,
        in_specs=[
            pl.BlockSpec((BE, H), lambda i: (i, 0)),
            pl.BlockSpec((BE, NIN), lambda i: (i, 0)),
            full((H, H)), full((NIN, H)), full((1, H)),
            full((H, H)), full((1, H)),
            full((H, H)), full((1, H)),
        ],
        out_specs=pl.BlockSpec((BE, H), lambda i: (i, 0)),
        out_shape=jax.ShapeDtypeStruct((N_EDGES, H), jnp.float32),
    )(xa, xb, w1a, w1b, W1_b.reshape(1, H), W2_w, W2_b.reshape(1, H),
      W3_w, W3_b.reshape(1, H))


def _scatter_body(hm_hbm, idx_hbm, zeros_hbm, out0_hbm, out1_hbm,
                  idx_v, rows_v, acc_sh, sem):
    cid = lax.axis_index("c")
    sid = lax.axis_index("s")
    wid = cid * NS + sid

    @pl.when(sid == 0)
    def _():
        pltpu.sync_copy(zeros_hbm, acc_sh)

    pltpu.sync_copy(idx_hbm.at[wid], idx_v)
    plsc.subcore_barrier()

    base = wid * EDGES_PER_WORKER

    def body(j, carry):
        pltpu.sync_copy(hm_hbm.at[pl.ds(base + j * CHUNK, CHUNK)], rows_v)
        pltpu.sync_copy(rows_v, acc_sh.at[idx_v.at[j]], add=True)
        return carry

    lax.fori_loop(0, NCHUNKS, body, 0)
    plsc.subcore_barrier()

    @pl.when((sid == 0) & (cid == 0))
    def _():
        pltpu.sync_copy(acc_sh, out0_hbm)

    @pl.when((sid == 0) & (cid == 1))
    def _():
        pltpu.sync_copy(acc_sh, out1_hbm)


@functools.cache
def _sc_segment_sum():
    return pl.kernel(
        _scatter_body,
        out_type=(jax.ShapeDtypeStruct((N_NODES, H), jnp.float32),
                  jax.ShapeDtypeStruct((N_NODES, H), jnp.float32)),
        mesh=plsc.VectorSubcoreMesh(core_axis_name="c", subcore_axis_name="s",
                                    num_cores=NC, num_subcores=NS),
        scratch_types=[
            pltpu.VMEM((NCHUNKS, CHUNK), jnp.int32),
            pltpu.VMEM((CHUNK, H), jnp.float32),
            pltpu.VMEM_SHARED((N_NODES, H), jnp.float32),
            pltpu.SemaphoreType.DMA,
        ],
    )


BN = 1000


def _node_body(hv_ref, p0_ref, p1_ref, d1w_ref, d1b_ref, d2w_ref, d2b_ref,
               g1_ref, b1_ref, g2_ref, b2_ref, out_ref):
    dh = (p0_ref[...] + p1_ref[...]) * (1.0 / SCALE)
    x = hv_ref[...] + dh
    mu = jnp.mean(x, axis=-1, keepdims=True)
    var = jnp.mean((x - mu) ** 2, axis=-1, keepdims=True)
    h = (x - mu) * lax.rsqrt(var + 1e-5) * g1_ref[...] + b1_ref[...]
    f = _gelu(jnp.dot(h, d1w_ref[...], preferred_element_type=jnp.float32)
              + d1b_ref[...])
    dh2 = jnp.dot(f, d2w_ref[...], preferred_element_type=jnp.float32) + d2b_ref[...]
    x2 = h + dh2
    mu2 = jnp.mean(x2, axis=-1, keepdims=True)
    var2 = jnp.mean((x2 - mu2) ** 2, axis=-1, keepdims=True)
    out_ref[...] = (x2 - mu2) * lax.rsqrt(var2 + 1e-5) * g2_ref[...] + b2_ref[...]


def _node_update(h_V, p0, p1, d1_w, d1_b, d2_w, d2_b, ln1_g, ln1_b, ln2_g, ln2_b):
    grid = (N_NODES // BN,)
    full = lambda s: pl.BlockSpec(s, lambda i: (0, 0))
    blk = pl.BlockSpec((BN, H), lambda i: (i, 0))
    return pl.pallas_call(
        _node_body,
        grid=grid,
        in_specs=[
            blk, blk, blk,
            full((H, 4 * H)), full((1, 4 * H)),
            full((4 * H, H)), full((1, H)),
            full((1, H)), full((1, H)), full((1, H)), full((1, H)),
        ],
        out_specs=blk,
        out_shape=jax.ShapeDtypeStruct((N_NODES, H), jnp.float32),
    )(h_V, p0, p1, d1_w, d1_b.reshape(1, 4 * H), d2_w, d2_b.reshape(1, H),
      ln1_g.reshape(1, H), ln1_b.reshape(1, H),
      ln2_g.reshape(1, H), ln2_b.reshape(1, H))


def kernel(h_V, h_E, edge_idx, W1_w, W1_b, W2_w, W2_b, W3_w, W3_b,
           d1_w, d1_b, d2_w, d2_b, ln1_g, ln1_b, ln2_g, ln2_b):
    h_message = _edge_mlp(h_E, W1_w, W1_b, W2_w, W2_b, W3_w, W3_b)
    src = edge_idx[0].astype(jnp.int32).reshape(NW, NCHUNKS, CHUNK)
    zeros = jnp.zeros((N_NODES, H), jnp.float32)
    p0, p1 = _sc_segment_sum()(h_message, src, zeros)
    return _node_update(h_V, p0, p1, d1_w, d1_b, d2_w, d2_b,
                        ln1_g, ln1_b, ln2_g, ln2_b)


# speedup vs baseline: 2.4877x; 1.1335x over previous
import functools

import jax
import jax.numpy as jnp
from jax import lax
from jax.experimental import pallas as pl
from jax.experimental.pallas import tpu as pltpu
from jax.experimental.pallas import tpu_sc as plsc

N_NODES = 10000
N_EDGES = 320000
H = 128
NIN = 16
SCALE = 30.0

NC = 2
NS = 16
NW = NC * NS
EDGES_PER_WORKER = N_EDGES // NW
CHUNK = 80
NCHUNKS = EDGES_PER_WORKER // CHUNK


def _gelu(x):
    return 0.5 * x * (1.0 + lax.erf(x * (2.0 ** -0.5)))


BE = 1280


def _edge_mlp_body(xa_ref, xb_ref, w1a_ref, w1b_ref, b1_ref, w2_ref, b2_ref,
                   w3_ref, b3_ref, out_ref):
    xa = xa_ref[...]
    xb = xb_ref[...]
    m = _gelu(jnp.dot(xa, w1a_ref[...], preferred_element_type=jnp.float32)
              + jnp.dot(xb, w1b_ref[...], preferred_element_type=jnp.float32)
              + b1_ref[...])
    m = _gelu(jnp.dot(m, w2_ref[...], preferred_element_type=jnp.float32)
              + b2_ref[...])
    out_ref[...] = (jnp.dot(m, w3_ref[...], preferred_element_type=jnp.float32)
                    + b3_ref[...])


def _edge_mlp(h_E, W1_w, W1_b, W2_w, W2_b, W3_w, W3_b):
    xa = h_E[:, :H]
    xb = h_E[:, H:]
    w1a = W1_w[:H]
    w1b = W1_w[H:]
    grid = (N_EDGES // BE,)
    full = lambda s: pl.BlockSpec(s, lambda i: (0, 0))
    return pl.pallas_call(
        _edge_mlp_body,
        grid=grid,
        in_specs=[
            pl.BlockSpec((BE, H), lambda i: (i, 0)),
            pl.BlockSpec((BE, NIN), lambda i: (i, 0)),
            full((H, H)), full((NIN, H)), full((1, H)),
            full((H, H)), full((1, H)),
            full((H, H)), full((1, H)),
        ],
        out_specs=pl.BlockSpec((BE, H), lambda i: (i, 0)),
        out_shape=jax.ShapeDtypeStruct((N_EDGES, H), jnp.float32),
    )(xa, xb, w1a, w1b, W1_b.reshape(1, H), W2_w, W2_b.reshape(1, H),
      W3_w, W3_b.reshape(1, H))


NBUF = 3


def _scatter_body(hm_hbm, idx_hbm, zeros_hbm, out0_hbm, out1_hbm,
                  idx_v, rows_v, acc_sh, sems):
    cid = lax.axis_index("c")
    sid = lax.axis_index("s")
    wid = cid * NS + sid

    @pl.when(sid == 0)
    def _():
        pltpu.sync_copy(zeros_hbm, acc_sh)

    pltpu.sync_copy(idx_hbm.at[wid], idx_v)
    plsc.subcore_barrier()

    base = wid * EDGES_PER_WORKER

    def _start(j, slot):
        pltpu.async_copy(hm_hbm.at[pl.ds(base + j * CHUNK, CHUNK)],
                         rows_v.at[slot], sems.at[slot])

    for j in range(NBUF - 1):
        _start(j, j)

    def body(j, carry):
        slot = lax.rem(j, NBUF)
        nxt = j + (NBUF - 1)

        @pl.when(nxt < NCHUNKS)
        def _():
            _start(nxt, lax.rem(nxt, NBUF))

        pltpu.make_async_copy(hm_hbm.at[pl.ds(base + j * CHUNK, CHUNK)],
                              rows_v.at[slot], sems.at[slot]).wait()
        pltpu.sync_copy(rows_v.at[slot], acc_sh.at[idx_v.at[j]], add=True)
        return carry

    lax.fori_loop(0, NCHUNKS, body, 0)
    plsc.subcore_barrier()

    @pl.when((sid == 0) & (cid == 0))
    def _():
        pltpu.sync_copy(acc_sh, out0_hbm)

    @pl.when((sid == 0) & (cid == 1))
    def _():
        pltpu.sync_copy(acc_sh, out1_hbm)


@functools.cache
def _sc_segment_sum():
    return pl.kernel(
        _scatter_body,
        out_type=(jax.ShapeDtypeStruct((N_NODES, H), jnp.float32),
                  jax.ShapeDtypeStruct((N_NODES, H), jnp.float32)),
        mesh=plsc.VectorSubcoreMesh(core_axis_name="c", subcore_axis_name="s",
                                    num_cores=NC, num_subcores=NS),
        scratch_types=[
            pltpu.VMEM((NCHUNKS, CHUNK), jnp.int32),
            pltpu.VMEM((NBUF, CHUNK, H), jnp.float32),
            pltpu.VMEM_SHARED((N_NODES, H), jnp.float32),
            pltpu.SemaphoreType.DMA((NBUF,)),
        ],
    )


BN = 1000


def _node_body(hv_ref, p0_ref, p1_ref, d1w_ref, d1b_ref, d2w_ref, d2b_ref,
               g1_ref, b1_ref, g2_ref, b2_ref, out_ref):
    dh = (p0_ref[...] + p1_ref[...]) * (1.0 / SCALE)
    x = hv_ref[...] + dh
    mu = jnp.mean(x, axis=-1, keepdims=True)
    var = jnp.mean((x - mu) ** 2, axis=-1, keepdims=True)
    h = (x - mu) * lax.rsqrt(var + 1e-5) * g1_ref[...] + b1_ref[...]
    f = _gelu(jnp.dot(h, d1w_ref[...], preferred_element_type=jnp.float32)
              + d1b_ref[...])
    dh2 = jnp.dot(f, d2w_ref[...], preferred_element_type=jnp.float32) + d2b_ref[...]
    x2 = h + dh2
    mu2 = jnp.mean(x2, axis=-1, keepdims=True)
    var2 = jnp.mean((x2 - mu2) ** 2, axis=-1, keepdims=True)
    out_ref[...] = (x2 - mu2) * lax.rsqrt(var2 + 1e-5) * g2_ref[...] + b2_ref[...]


def _node_update(h_V, p0, p1, d1_w, d1_b, d2_w, d2_b, ln1_g, ln1_b, ln2_g, ln2_b):
    grid = (N_NODES // BN,)
    full = lambda s: pl.BlockSpec(s, lambda i: (0, 0))
    blk = pl.BlockSpec((BN, H), lambda i: (i, 0))
    return pl.pallas_call(
        _node_body,
        grid=grid,
        in_specs=[
            blk, blk, blk,
            full((H, 4 * H)), full((1, 4 * H)),
            full((4 * H, H)), full((1, H)),
            full((1, H)), full((1, H)), full((1, H)), full((1, H)),
        ],
        out_specs=blk,
        out_shape=jax.ShapeDtypeStruct((N_NODES, H), jnp.float32),
    )(h_V, p0, p1, d1_w, d1_b.reshape(1, 4 * H), d2_w, d2_b.reshape(1, H),
      ln1_g.reshape(1, H), ln1_b.reshape(1, H),
      ln2_g.reshape(1, H), ln2_b.reshape(1, H))


def kernel(h_V, h_E, edge_idx, W1_w, W1_b, W2_w, W2_b, W3_w, W3_b,
           d1_w, d1_b, d2_w, d2_b, ln1_g, ln1_b, ln2_g, ln2_b):
    h_message = _edge_mlp(h_E, W1_w, W1_b, W2_w, W2_b, W3_w, W3_b)
    src = edge_idx[0].astype(jnp.int32).reshape(NW, NCHUNKS, CHUNK)
    zeros = jnp.zeros((N_NODES, H), jnp.float32)
    p0, p1 = _sc_segment_sum()(h_message, src, zeros)
    return _node_update(h_V, p0, p1, d1_w, d1_b, d2_w, d2_b,
                        ln1_g, ln1_b, ln2_g, ln2_b)
